# Initial kernel scaffold; baseline (speedup 1.0000x reference)
import jax
import jax.numpy as jnp
from jax.experimental import pallas as pl


def kernel(X, adj, W1, b1, W2, b2, W3, b3):
    raise NotImplementedError("write your pallas kernel here")


# speedup vs baseline: 8.8582x; 8.8582x over previous
import functools

import jax
import jax.numpy as jnp
from jax import lax
from jax.experimental import pallas as pl
from jax.experimental.pallas import tpu as pltpu
from jax.experimental.pallas import tpu_sc as plsc

N = 10000
E = 320000
D = 128

NC = 2
NS = 16
C = 128
CPT = 80
TOTAL_CHUNKS = NC * NS * CPT
E_PAD = TOTAL_CHUNKS * C
PAD = E_PAD - E
NPAD = 10240
RPT = NPAD // NS
ZR = 64


def _segsum_sc(x, src2d, dst2d, zeros_blk):
    mesh = plsc.VectorSubcoreMesh(core_axis_name="c", subcore_axis_name="s")

    @functools.partial(
        pl.kernel,
        mesh=mesh,
        out_type=jax.ShapeDtypeStruct((NC, NPAD, D), jnp.float32),
        scratch_types=[
            pltpu.VMEM((CPT, C), jnp.int32),
            pltpu.VMEM((CPT, C), jnp.int32),
            pltpu.VMEM((C, D), jnp.float32),
            pltpu.VMEM((ZR, D), jnp.float32),
            pltpu.VMEM_SHARED((NPAD, D), jnp.float32),
            pltpu.SemaphoreType.DMA,
        ],
    )
    def seg_kernel(x_hbm, src_hbm, dst_hbm, z_hbm, out_hbm,
                   srcbuf, dstbuf, rbuf, zbuf, acc, gsem):
        c = lax.axis_index("c")
        s = lax.axis_index("s")
        row0 = s * RPT
        pltpu.sync_copy(z_hbm, zbuf)
        for r in range(RPT // ZR):
            pltpu.sync_copy(zbuf, acc.at[pl.ds(row0 + r * ZR, ZR)])
        plsc.subcore_barrier()
        chunk0 = (c * NS + s) * CPT
        pltpu.sync_copy(src_hbm.at[pl.ds(chunk0, CPT)], srcbuf)
        pltpu.sync_copy(dst_hbm.at[pl.ds(chunk0, CPT)], dstbuf)

        def chunk_body(j, carry):
            pltpu.async_copy(x_hbm.at[srcbuf.at[j]], rbuf, gsem).wait()
            pltpu.sync_copy(rbuf, acc.at[dstbuf.at[j]], add=True)
            return carry

        lax.fori_loop(0, CPT, chunk_body, 0)
        plsc.subcore_barrier()
        pltpu.sync_copy(acc.at[pl.ds(row0, RPT)],
                        out_hbm.at[c, pl.ds(row0, RPT)])

    return seg_kernel(x, src2d, dst2d, zeros_blk)


def _lrelu(x):
    return jnp.where(x > 0, x, 0.01 * x)


_BR = 640


def _stage1(s0, s1, w1, b1, w3, b3):
    def body(s0_ref, s1_ref, w1_ref, b1_ref, w3_ref, b3_ref, y1_ref, p3_ref):
        sblk = s0_ref[...] + s1_ref[...]
        h1 = jnp.dot(sblk, w1_ref[...], preferred_element_type=jnp.float32,
                     precision=lax.Precision.HIGHEST) + b1_ref[...]
        y1_ref[...] = _lrelu(h1)
        p3_ref[...] = jnp.dot(sblk, w3_ref[...],
                              preferred_element_type=jnp.float32,
                              precision=lax.Precision.HIGHEST) + b3_ref[...]

    row = pl.BlockSpec((_BR, D), lambda i: (i, 0))
    full = pl.BlockSpec((D, D), lambda i: (0, 0))
    vec = pl.BlockSpec((1, D), lambda i: (0, 0))
    return pl.pallas_call(
        body,
        grid=(NPAD // _BR,),
        in_specs=[row, row, full, vec, full, vec],
        out_specs=[row, row],
        out_shape=[jax.ShapeDtypeStruct((NPAD, D), jnp.float32),
                   jax.ShapeDtypeStruct((NPAD, D), jnp.float32)],
    )(s0, s1, w1, b1.reshape(1, D), w3, b3.reshape(1, D))


def _stage2(t0, t1, w2, b2, p3):
    def body(t0_ref, t1_ref, w2_ref, b2_ref, p3_ref, o_ref):
        tblk = t0_ref[...] + t1_ref[...]
        h2 = jnp.dot(tblk, w2_ref[...], preferred_element_type=jnp.float32,
                     precision=lax.Precision.HIGHEST) + b2_ref[...]
        o_ref[...] = _lrelu(_lrelu(h2) + p3_ref[...])

    row = pl.BlockSpec((_BR, D), lambda i: (i, 0))
    full = pl.BlockSpec((D, D), lambda i: (0, 0))
    vec = pl.BlockSpec((1, D), lambda i: (0, 0))
    return pl.pallas_call(
        body,
        grid=(NPAD // _BR,),
        in_specs=[row, row, full, vec, row],
        out_specs=row,
        out_shape=jax.ShapeDtypeStruct((NPAD, D), jnp.float32),
    )(t0, t1, w2, b2.reshape(1, D), p3)


def kernel(X, adj, W1, b1, W2, b2, W3, b3):
    src = adj[0]
    dst = adj[1]
    ar = jnp.arange(PAD, dtype=jnp.int32)
    pad_src = (ar * 1009) % N
    pad_dst = N + ar % (NPAD - N)
    src2d = jnp.concatenate([src, pad_src]).reshape(TOTAL_CHUNKS, C)
    dst2d = jnp.concatenate([dst, pad_dst]).reshape(TOTAL_CHUNKS, C)
    zeros_blk = jnp.zeros((ZR, D), jnp.float32)

    parts = _segsum_sc(X, src2d, dst2d, zeros_blk)
    y1, p3 = _stage1(parts[0], parts[1], W1, b1, W3, b3)
    parts2 = _segsum_sc(y1, src2d, dst2d, zeros_blk)
    out = _stage2(parts2[0], parts2[1], W2, b2, p3)
    return out[:N]


# speedup vs baseline: 11.1130x; 1.2546x over previous
import functools

import jax
import jax.numpy as jnp
from jax import lax
from jax.experimental import pallas as pl
from jax.experimental.pallas import tpu as pltpu
from jax.experimental.pallas import tpu_sc as plsc

N = 10000
E = 320000
D = 128

NC = 2
NS = 16
C = 128
CPT = 80
SCH = 40
TOTAL_CHUNKS = NC * NS * CPT
E_PAD = TOTAL_CHUNKS * C
PAD = E_PAD - E
NPAD = 10240
RPT = NPAD // NS
ZR = 32


def _segsum_sc(x, src2d, dst2d, zeros_blk):
    mesh = plsc.VectorSubcoreMesh(core_axis_name="c", subcore_axis_name="s")

    @functools.partial(
        pl.kernel,
        mesh=mesh,
        out_type=jax.ShapeDtypeStruct((NC, NPAD, D), jnp.float32),
        scratch_types=[
            pltpu.VMEM((SCH, C), jnp.int32),
            pltpu.VMEM((SCH, C), jnp.int32),
            pltpu.VMEM((C, D), jnp.float32),
            pltpu.VMEM((C, D), jnp.float32),
            pltpu.VMEM_SHARED((NPAD, D), jnp.float32),
            pltpu.SemaphoreType.DMA,
            pltpu.SemaphoreType.DMA,
            pltpu.SemaphoreType.DMA,
            pltpu.SemaphoreType.DMA,
        ],
    )
    def seg_kernel(x_hbm, src_hbm, dst_hbm, z_hbm, out_hbm,
                   srcbuf, dstbuf, rb0, rb1, acc,
                   gsem0, gsem1, ssem0, ssem1):
        c = lax.axis_index("c")
        s = lax.axis_index("s")
        row0 = s * RPT
        pltpu.sync_copy(z_hbm, rb0.at[pl.ds(0, ZR)])
        for r in range(RPT // ZR):
            pltpu.sync_copy(rb0.at[pl.ds(0, ZR)],
                            acc.at[pl.ds(row0 + r * ZR, ZR)])
        plsc.subcore_barrier()
        chunk0 = (c * NS + s) * CPT

        def g_start(j, buf, sem):
            pltpu.async_copy(x_hbm.at[srcbuf.at[j]], buf, sem)

        def g_wait(buf, sem):
            pltpu.make_async_copy(x_hbm.at[srcbuf.at[0]], buf, sem).wait()

        def s_start(j, buf, sem):
            pltpu.async_copy(buf, acc.at[dstbuf.at[j]], sem, add=True)

        def s_wait(buf, sem):
            pltpu.make_async_copy(buf, acc.at[dstbuf.at[0]], sem).wait()

        for st in range(CPT // SCH):
            base = chunk0 + st * SCH
            pltpu.sync_copy(src_hbm.at[pl.ds(base, SCH)], srcbuf)
            pltpu.sync_copy(dst_hbm.at[pl.ds(base, SCH)], dstbuf)
            g_start(0, rb0, gsem0)

            def chunk_body(i, carry):
                j0 = 2 * i
                g_wait(rb0, gsem0)
                g_start(j0 + 1, rb1, gsem1)
                s_start(j0, rb0, ssem0)
                g_wait(rb1, gsem1)
                s_wait(rb0, ssem0)

                @pl.when(j0 + 2 < SCH)
                def _():
                    g_start(j0 + 2, rb0, gsem0)

                s_start(j0 + 1, rb1, ssem1)
                s_wait(rb1, ssem1)
                return carry

            lax.fori_loop(0, SCH // 2, chunk_body, 0)
        plsc.subcore_barrier()
        pltpu.sync_copy(acc.at[pl.ds(row0, RPT)],
                        out_hbm.at[c, pl.ds(row0, RPT)])

    return seg_kernel(x, src2d, dst2d, zeros_blk)


def _lrelu(x):
    return jnp.where(x > 0, x, 0.01 * x)


_BR = 640


def _stage1(s0, s1, w1, b1, w3, b3):
    def body(s0_ref, s1_ref, w1_ref, b1_ref, w3_ref, b3_ref, y1_ref, p3_ref):
        sblk = s0_ref[...] + s1_ref[...]
        h1 = jnp.dot(sblk, w1_ref[...], preferred_element_type=jnp.float32,
                     precision=lax.Precision.HIGHEST) + b1_ref[...]
        y1_ref[...] = _lrelu(h1)
        p3_ref[...] = jnp.dot(sblk, w3_ref[...],
                              preferred_element_type=jnp.float32,
                              precision=lax.Precision.HIGHEST) + b3_ref[...]

    row = pl.BlockSpec((_BR, D), lambda i: (i, 0))
    full = pl.BlockSpec((D, D), lambda i: (0, 0))
    vec = pl.BlockSpec((1, D), lambda i: (0, 0))
    return pl.pallas_call(
        body,
        grid=(NPAD // _BR,),
        in_specs=[row, row, full, vec, full, vec],
        out_specs=[row, row],
        out_shape=[jax.ShapeDtypeStruct((NPAD, D), jnp.float32),
                   jax.ShapeDtypeStruct((NPAD, D), jnp.float32)],
    )(s0, s1, w1, b1.reshape(1, D), w3, b3.reshape(1, D))


def _stage2(t0, t1, w2, b2, p3):
    def body(t0_ref, t1_ref, w2_ref, b2_ref, p3_ref, o_ref):
        tblk = t0_ref[...] + t1_ref[...]
        h2 = jnp.dot(tblk, w2_ref[...], preferred_element_type=jnp.float32,
                     precision=lax.Precision.HIGHEST) + b2_ref[...]
        o_ref[...] = _lrelu(_lrelu(h2) + p3_ref[...])

    row = pl.BlockSpec((_BR, D), lambda i: (i, 0))
    full = pl.BlockSpec((D, D), lambda i: (0, 0))
    vec = pl.BlockSpec((1, D), lambda i: (0, 0))
    return pl.pallas_call(
        body,
        grid=(NPAD // _BR,),
        in_specs=[row, row, full, vec, row],
        out_specs=row,
        out_shape=jax.ShapeDtypeStruct((NPAD, D), jnp.float32),
    )(t0, t1, w2, b2.reshape(1, D), p3)


def kernel(X, adj, W1, b1, W2, b2, W3, b3):
    src = adj[0]
    dst = adj[1]
    ar = jnp.arange(PAD, dtype=jnp.int32)
    pad_src = (ar * 1009) % N
    pad_dst = N + ar % (NPAD - N)
    src2d = jnp.concatenate([src, pad_src]).reshape(TOTAL_CHUNKS, C)
    dst2d = jnp.concatenate([dst, pad_dst]).reshape(TOTAL_CHUNKS, C)
    zeros_blk = jnp.zeros((ZR, D), jnp.float32)

    parts = _segsum_sc(X, src2d, dst2d, zeros_blk)
    y1, p3 = _stage1(parts[0], parts[1], W1, b1, W3, b3)
    parts2 = _segsum_sc(y1, src2d, dst2d, zeros_blk)
    out = _stage2(parts2[0], parts2[1], W2, b2, p3)
    return out[:N]


# speedup vs baseline: 11.9383x; 1.0743x over previous
import functools

import jax
import jax.numpy as jnp
from jax import lax
from jax.experimental import pallas as pl
from jax.experimental.pallas import tpu as pltpu
from jax.experimental.pallas import tpu_sc as plsc

N = 10000
E = 320000
D = 128

NC = 2
NS = 16
C = 128
NCHUNK = E // C
CPT = NCHUNK // (NC * NS)
LEFTOVER = NCHUNK - CPT * NC * NS
SCH = 26
NACC = 10112
RPT = NACC // NS
ZR = 40


def _segsum_sc(x, src1d, dst1d, zeros_blk):
    mesh = plsc.VectorSubcoreMesh(core_axis_name="c", subcore_axis_name="s")

    @functools.partial(
        pl.kernel,
        mesh=mesh,
        out_type=jax.ShapeDtypeStruct((NC, N, D), jnp.float32),
        scratch_types=[
            pltpu.VMEM((SCH * C,), jnp.int32),
            pltpu.VMEM((SCH * C,), jnp.int32),
            pltpu.VMEM((C, D), jnp.float32),
            pltpu.VMEM((C, D), jnp.float32),
            pltpu.VMEM_SHARED((NACC, D), jnp.float32),
            pltpu.SemaphoreType.DMA,
            pltpu.SemaphoreType.DMA,
            pltpu.SemaphoreType.DMA,
            pltpu.SemaphoreType.DMA,
        ],
    )
    def seg_kernel(x_hbm, src_hbm, dst_hbm, z_hbm, out_hbm,
                   srcbuf, dstbuf, rb0, rb1, acc,
                   gsem0, gsem1, ssem0, ssem1):
        c = lax.axis_index("c")
        s = lax.axis_index("s")
        wid = c * NS + s
        row0 = s * RPT
        pltpu.sync_copy(z_hbm, rb0.at[pl.ds(0, ZR)])
        for r in range(RPT // ZR):
            pltpu.sync_copy(rb0.at[pl.ds(0, ZR)],
                            acc.at[pl.ds(row0 + r * ZR, ZR)])
        pltpu.sync_copy(rb0.at[pl.ds(0, RPT % ZR)],
                        acc.at[pl.ds(row0 + (RPT // ZR) * ZR, RPT % ZR)])
        plsc.subcore_barrier()
        chunk0 = wid * CPT

        def g_start(j, buf, sem):
            pltpu.async_copy(x_hbm.at[srcbuf.at[pl.ds(j * C, C)]], buf, sem)

        def g_wait(buf, sem):
            pltpu.make_async_copy(x_hbm.at[srcbuf.at[pl.ds(0, C)]],
                                  buf, sem).wait()

        def s_start(j, buf, sem):
            pltpu.async_copy(buf, acc.at[dstbuf.at[pl.ds(j * C, C)]], sem,
                             add=True)

        def s_wait(buf, sem):
            pltpu.make_async_copy(buf, acc.at[dstbuf.at[pl.ds(0, C)]],
                                  sem).wait()

        for st in range(CPT // SCH):
            base = (chunk0 + st * SCH) * C
            pltpu.sync_copy(src_hbm.at[pl.ds(base, SCH * C)], srcbuf)
            pltpu.sync_copy(dst_hbm.at[pl.ds(base, SCH * C)], dstbuf)
            g_start(0, rb0, gsem0)

            def chunk_body(i, carry):
                j0 = 2 * i
                g_wait(rb0, gsem0)
                g_start(j0 + 1, rb1, gsem1)
                s_start(j0, rb0, ssem0)
                g_wait(rb1, gsem1)
                s_wait(rb0, ssem0)

                @pl.when(j0 + 2 < SCH)
                def _():
                    g_start(j0 + 2, rb0, gsem0)

                s_start(j0 + 1, rb1, ssem1)
                s_wait(rb1, ssem1)
                return carry

            lax.fori_loop(0, SCH // 2, chunk_body, 0)

        @pl.when(wid < LEFTOVER)
        def _():
            off = (CPT * NC * NS + wid) * C
            pltpu.sync_copy(src_hbm.at[pl.ds(off, C)], srcbuf.at[pl.ds(0, C)])
            pltpu.sync_copy(dst_hbm.at[pl.ds(off, C)], dstbuf.at[pl.ds(0, C)])
            g_start(0, rb0, gsem0)
            g_wait(rb0, gsem0)
            s_start(0, rb0, ssem0)
            s_wait(rb0, ssem0)

        plsc.subcore_barrier()
        @pl.when(s < NS - 1)
        def _():
            pltpu.sync_copy(acc.at[pl.ds(row0, RPT)],
                            out_hbm.at[c, pl.ds(row0, RPT)])

        @pl.when(s == NS - 1)
        def _():
            pltpu.sync_copy(acc.at[pl.ds(row0, N - (NS - 1) * RPT)],
                            out_hbm.at[c, pl.ds(row0, N - (NS - 1) * RPT)])

    return seg_kernel(x, src1d, dst1d, zeros_blk)


def _lrelu(x):
    return jnp.where(x > 0, x, 0.01 * x)


_BR = 2000


def _stage1(parts, w1, b1, w3, b3):
    def body(p_ref, w1_ref, b1_ref, w3_ref, b3_ref, y1_ref, p3_ref):
        sblk = p_ref[0] + p_ref[1]
        h1 = jnp.dot(sblk, w1_ref[...], preferred_element_type=jnp.float32,
                     precision=lax.Precision.HIGHEST) + b1_ref[...]
        y1_ref[...] = _lrelu(h1)
        p3_ref[...] = jnp.dot(sblk, w3_ref[...],
                              preferred_element_type=jnp.float32,
                              precision=lax.Precision.HIGHEST) + b3_ref[...]

    pair = pl.BlockSpec((NC, _BR, D), lambda i: (0, i, 0))
    row = pl.BlockSpec((_BR, D), lambda i: (i, 0))
    full = pl.BlockSpec((D, D), lambda i: (0, 0))
    vec = pl.BlockSpec((1, D), lambda i: (0, 0))
    return pl.pallas_call(
        body,
        grid=(N // _BR,),
        in_specs=[pair, full, vec, full, vec],
        out_specs=[row, row],
        out_shape=[jax.ShapeDtypeStruct((N, D), jnp.float32),
                   jax.ShapeDtypeStruct((N, D), jnp.float32)],
    )(parts, w1, b1.reshape(1, D), w3, b3.reshape(1, D))


def _stage2(parts, w2, b2, p3):
    def body(p_ref, w2_ref, b2_ref, p3_ref, o_ref):
        tblk = p_ref[0] + p_ref[1]
        h2 = jnp.dot(tblk, w2_ref[...], preferred_element_type=jnp.float32,
                     precision=lax.Precision.HIGHEST) + b2_ref[...]
        o_ref[...] = _lrelu(_lrelu(h2) + p3_ref[...])

    pair = pl.BlockSpec((NC, _BR, D), lambda i: (0, i, 0))
    row = pl.BlockSpec((_BR, D), lambda i: (i, 0))
    full = pl.BlockSpec((D, D), lambda i: (0, 0))
    vec = pl.BlockSpec((1, D), lambda i: (0, 0))
    return pl.pallas_call(
        body,
        grid=(N // _BR,),
        in_specs=[pair, full, vec, row],
        out_specs=row,
        out_shape=jax.ShapeDtypeStruct((N, D), jnp.float32),
    )(parts, w2, b2.reshape(1, D), p3)


def kernel(X, adj, W1, b1, W2, b2, W3, b3):
    src1d = adj[0]
    dst1d = adj[1]
    zeros_blk = jnp.zeros((ZR, D), jnp.float32)

    parts = _segsum_sc(X, src1d, dst1d, zeros_blk)
    y1, p3 = _stage1(parts, W1, b1, W3, b3)
    parts2 = _segsum_sc(y1, src1d, dst1d, zeros_blk)
    return _stage2(parts2, W2, b2, p3)
